# baseline (speedup 1.0000x reference)
import functools

import jax
import jax.numpy as jnp
from jax import lax
from jax.experimental import pallas as pl
from jax.experimental.pallas import tpu as pltpu
from jax.experimental.pallas import tpu_sc as plsc

T = 4096
D = 1024
F = 4096
E = 8
CAP = 1280
EH = E // 2
NR = E * CAP
NRH = EH * CAP
PADR = CAP
NRP = NRH + PADR
TRASH = NRH
FT = 1024
NFT = F // FT

_SQRT2 = 1.4142135623730951


def _router_body(x_ref, wr_ref, meta_ref, aux_ref):
    xv = x_ref[...]
    wr = wr_ref[...]
    lg = lax.dot_general(wr, xv, (((1,), (1,)), ((), ())),
                         preferred_element_type=jnp.float32)
    m = jnp.max(lg, axis=0, keepdims=True)
    ex = jnp.exp(lg - m)
    probs = ex / jnp.sum(ex, axis=0, keepdims=True)

    iota = lax.broadcasted_iota(jnp.int32, (E, T), 0)
    m1 = jnp.max(probs, axis=0, keepdims=True)
    i1 = jnp.min(jnp.where(probs == m1, iota, E), axis=0, keepdims=True)
    pmask = iota == i1
    pm = jnp.where(pmask, -jnp.inf, probs)
    m2 = jnp.max(pm, axis=0, keepdims=True)
    i2 = jnp.min(jnp.where(pm == m2, iota, E), axis=0, keepdims=True)

    denom = jnp.clip(m1 + m2, 1e-9, None)
    g1 = m1 / denom
    g2 = m2 / denom

    oh1 = (iota == i1).astype(jnp.float32)
    oh2 = (iota == i2).astype(jnp.float32)
    gt = oh1 + oh2

    c = gt
    k = 1
    while k < T:
        shifted = jnp.concatenate(
            [jnp.zeros((E, k), jnp.float32), c[:, : T - k]], axis=1)
        c = c + shifted
        k *= 2
    excl = c - gt

    r1 = jnp.sum(oh1 * excl, axis=0, keepdims=True) + 1.0
    r2 = jnp.sum(oh2 * excl, axis=0, keepdims=True) + 1.0
    acc1 = r1 <= float(CAP)
    acc2 = r2 <= float(CAP)
    i1f = i1.astype(jnp.float32)
    i2f = i2.astype(jnp.float32)
    d1 = i1f * float(CAP) + r1 - 1.0
    d2 = i2f * float(CAP) + r2 - 1.0
    lo1 = i1 < EH
    lo2 = i2 < EH
    d1h0 = jnp.where(acc1 & lo1, d1, float(TRASH))
    d1h1 = jnp.where(acc1 & ~lo1, d1 - float(EH * CAP), float(TRASH))
    d2h0 = jnp.where(acc2 & lo2, d2, float(TRASH))
    d2h1 = jnp.where(acc2 & ~lo2, d2 - float(EH * CAP), float(TRASH))
    ga0 = jnp.where(acc1 & lo1, g1, 0.0)
    ga1 = jnp.where(acc1 & ~lo1, g1, 0.0)
    gb0 = jnp.where(acc2 & lo2, g2, 0.0)
    gb1 = jnp.where(acc2 & ~lo2, g2, 0.0)
    dmask = 1.0 - jnp.maximum(acc1.astype(jnp.float32), acc2.astype(jnp.float32))

    meta_ref[...] = jnp.concatenate(
        [d1h0, d1h1, d2h0, d2h1, ga0, ga1, gb0, gb1, dmask,
         jnp.zeros((7, T), jnp.float32)], axis=0)

    imp = jnp.mean(probs, axis=1, keepdims=True)
    cnt = jnp.sum(gt, axis=1, keepdims=True)
    load = cnt / jnp.clip(jnp.sum(cnt), 1e-9, None)
    aux_ref[...] = jnp.reshape(float(E) * jnp.sum(imp * load), (1, 1))


def _router(x, wr):
    return pl.pallas_call(
        _router_body,
        out_shape=[
            jax.ShapeDtypeStruct((16, T), jnp.float32),
            jax.ShapeDtypeStruct((1, 1), jnp.float32),
        ],
    )(x, wr)


_NW = 32
_TPW = T // _NW
_CH = 64


def _dispatch_body(x_hbm, d0_hbm, d1_hbm, xe_hbm, xbuf, i0, i1, sem):
    wid = lax.axis_index("s") * 2 + lax.axis_index("c")
    for cch in range(_TPW // _CH):
        base = wid * _TPW + cch * _CH
        pltpu.sync_copy(x_hbm.at[pl.ds(base, _CH)], xbuf)
        pltpu.sync_copy(d0_hbm.at[pl.ds(base, _CH)], i0)
        pltpu.sync_copy(d1_hbm.at[pl.ds(base, _CH)], i1)
        pltpu.async_copy(xbuf, xe_hbm.at[i0], sem).wait()
        pltpu.async_copy(xbuf, xe_hbm.at[i1], sem).wait()


def _dispatch(x, d0, d1):
    mesh = plsc.VectorSubcoreMesh(core_axis_name="c", subcore_axis_name="s")
    return pl.kernel(
        _dispatch_body,
        mesh=mesh,
        out_type=jax.ShapeDtypeStruct((NRP, D), jnp.float32),
        scratch_types=[
            pltpu.VMEM((_CH, D), jnp.float32),
            pltpu.VMEM((_CH,), jnp.int32),
            pltpu.VMEM((_CH,), jnp.int32),
            pltpu.SemaphoreType.DMA,
        ],
    )(x, d0, d1)


def _mlp_body(xe_ref, w1_ref, b1_ref, w2_ref, b2_ref, ye_ref):
    ff = pl.program_id(1)

    @pl.when(ff == 0)
    def _init():
        ye_ref[...] = jnp.broadcast_to(b2_ref[0], (CAP, D))

    xv = xe_ref[...]
    w1 = w1_ref[0]
    h = lax.dot_general(xv, w1, (((1,), (1,)), ((), ())),
                        preferred_element_type=jnp.float32)
    h = h + b1_ref[0]
    h = 0.5 * h * (1.0 + lax.erf(h / _SQRT2))
    w2 = w2_ref[0]
    y = lax.dot_general(h, w2, (((1,), (1,)), ((), ())),
                        preferred_element_type=jnp.float32)
    ye_ref[...] += y


def _mlp(xe, w1, b1, w2, b2, eoff):
    return pl.pallas_call(
        _mlp_body,
        grid=(EH, NFT),
        in_specs=[
            pl.BlockSpec((CAP, D), lambda e, ff: (e, 0)),
            pl.BlockSpec((1, FT, D), lambda e, ff, o=eoff: (e + o, ff, 0)),
            pl.BlockSpec((1, 1, FT), lambda e, ff, o=eoff: (e + o, 0, ff)),
            pl.BlockSpec((1, D, FT), lambda e, ff, o=eoff: (e + o, 0, ff)),
            pl.BlockSpec((1, 1, D), lambda e, ff, o=eoff: (e + o, 0, 0)),
        ],
        out_specs=pl.BlockSpec((CAP, D), lambda e, ff: (e, 0)),
        out_shape=jax.ShapeDtypeStruct((NRP, D), jnp.float32),
    )(xe, w1, b1.reshape(E, 1, F), w2, b2.reshape(E, 1, D))


def _gather_body(ye_hbm, r0_hbm, r1_hbm, a_hbm, b_hbm, buf, idx, sem):
    wid = lax.axis_index("s") * 2 + lax.axis_index("c")
    for cch in range(_TPW // _CH):
        base = wid * _TPW + cch * _CH
        pltpu.sync_copy(r0_hbm.at[pl.ds(base, _CH)], idx)
        pltpu.async_copy(ye_hbm.at[idx], buf, sem).wait()
        pltpu.sync_copy(buf, a_hbm.at[pl.ds(base, _CH)])
        pltpu.sync_copy(r1_hbm.at[pl.ds(base, _CH)], idx)
        pltpu.async_copy(ye_hbm.at[idx], buf, sem).wait()
        pltpu.sync_copy(buf, b_hbm.at[pl.ds(base, _CH)])


def _gather2(ye, r0, r1):
    mesh = plsc.VectorSubcoreMesh(core_axis_name="c", subcore_axis_name="s")
    return pl.kernel(
        _gather_body,
        mesh=mesh,
        out_type=[
            jax.ShapeDtypeStruct((T, D), jnp.float32),
            jax.ShapeDtypeStruct((T, D), jnp.float32),
        ],
        scratch_types=[
            pltpu.VMEM((_CH, D), jnp.float32),
            pltpu.VMEM((_CH,), jnp.int32),
            pltpu.SemaphoreType.DMA,
        ],
    )(ye, r0, r1)


def _combine_body(x_ref, a0_ref, a1_ref, b0_ref, b1_ref,
                  g_refs, dm_ref, out_ref):
    za = jnp.zeros_like(x_ref[...])
    acc = dm_ref[...] * x_ref[...]
    for g_ref, src in zip(g_refs, (a0_ref, a1_ref, b0_ref, b1_ref)):
        g = g_ref[...]
        acc = acc + jnp.where(g > 0.0, g * src[...], za)
    out_ref[...] = acc


def _combine(x, a0, a1, b0, b1, gs, dm):
    nt = 32
    bt = T // nt
    row = pl.BlockSpec((bt, D), lambda i: (i, 0))
    col = pl.BlockSpec((bt, 1), lambda i: (i, 0))

    def body(x_ref, a0_ref, a1_ref, b0_ref, b1_ref,
             g0_ref, g1_ref, g2_ref, g3_ref, dm_ref, out_ref):
        _combine_body(x_ref, a0_ref, a1_ref, b0_ref, b1_ref,
                      (g0_ref, g1_ref, g2_ref, g3_ref), dm_ref, out_ref)

    return pl.pallas_call(
        body,
        grid=(nt,),
        in_specs=[row, row, row, row, row, col, col, col, col, col],
        out_specs=row,
        out_shape=jax.ShapeDtypeStruct((T, D), jnp.float32),
    )(x, a0, a1, b0, b1, *gs, dm)


def kernel(x, Wr, W1, b1, W2, b2):
    meta, aux = _router(x, Wr)
    d0h0 = meta[0].astype(jnp.int32)
    d0h1 = meta[1].astype(jnp.int32)
    d1h0 = meta[2].astype(jnp.int32)
    d1h1 = meta[3].astype(jnp.int32)
    xe0 = _dispatch(x, d0h0, d1h0)
    xe1 = _dispatch(x, d0h1, d1h1)
    ye0 = _mlp(xe0, W1, b1, W2, b2, 0)
    ye1 = _mlp(xe1, W1, b1, W2, b2, EH)
    a0, b0 = _gather2(ye0, d0h0, d1h0)
    a1, b1g = _gather2(ye1, d0h1, d1h1)
    gs = tuple(meta[4 + i].reshape(T, 1) for i in range(4))
    dm = meta[8].reshape(T, 1)
    out = _combine(x, a0, a1, b0, b1g, gs, dm)
    return out, aux[0, 0]


# speedup vs baseline: 2.7895x; 2.7895x over previous
import functools

import jax
import jax.numpy as jnp
from jax import lax
from jax.experimental import pallas as pl
from jax.experimental.pallas import tpu as pltpu
from jax.experimental.pallas import tpu_sc as plsc

T = 4096
D = 1024
F = 4096
E = 8
CAP = 1280
EH = E // 2
NR = E * CAP
NRH = EH * CAP
PADR = CAP
NRP = NRH + PADR
TRASH = NRH
FT = 1024
NFT = F // FT

_SQRT2 = 1.4142135623730951


def _router_body(x_ref, wr_ref, meta_ref, aux_ref):
    xv = x_ref[...]
    wr = wr_ref[...]
    lg = lax.dot_general(wr, xv, (((1,), (1,)), ((), ())),
                         preferred_element_type=jnp.float32)
    m = jnp.max(lg, axis=0, keepdims=True)
    ex = jnp.exp(lg - m)
    probs = ex / jnp.sum(ex, axis=0, keepdims=True)

    iota = lax.broadcasted_iota(jnp.int32, (E, T), 0)
    m1 = jnp.max(probs, axis=0, keepdims=True)
    i1 = jnp.min(jnp.where(probs == m1, iota, E), axis=0, keepdims=True)
    pmask = iota == i1
    pm = jnp.where(pmask, -jnp.inf, probs)
    m2 = jnp.max(pm, axis=0, keepdims=True)
    i2 = jnp.min(jnp.where(pm == m2, iota, E), axis=0, keepdims=True)

    denom = jnp.clip(m1 + m2, 1e-9, None)
    g1 = m1 / denom
    g2 = m2 / denom

    oh1 = (iota == i1).astype(jnp.float32)
    oh2 = (iota == i2).astype(jnp.float32)
    gt = oh1 + oh2

    c = gt
    k = 1
    while k < T:
        shifted = jnp.concatenate(
            [jnp.zeros((E, k), jnp.float32), c[:, : T - k]], axis=1)
        c = c + shifted
        k *= 2
    excl = c - gt

    r1 = jnp.sum(oh1 * excl, axis=0, keepdims=True) + 1.0
    r2 = jnp.sum(oh2 * excl, axis=0, keepdims=True) + 1.0
    acc1 = r1 <= float(CAP)
    acc2 = r2 <= float(CAP)
    i1f = i1.astype(jnp.float32)
    i2f = i2.astype(jnp.float32)
    d1 = i1f * float(CAP) + r1 - 1.0
    d2 = i2f * float(CAP) + r2 - 1.0
    lo1 = i1 < EH
    lo2 = i2 < EH
    tl = lax.broadcasted_iota(jnp.int32, (1, T), 1)
    trashf = float(TRASH) + lax.rem(tl, PADR).astype(jnp.float32)
    d1h0 = jnp.where(acc1 & lo1, d1, trashf)
    d1h1 = jnp.where(acc1 & ~lo1, d1 - float(EH * CAP), trashf)
    d2h0 = jnp.where(acc2 & lo2, d2, trashf)
    d2h1 = jnp.where(acc2 & ~lo2, d2 - float(EH * CAP), trashf)
    ga0 = jnp.where(acc1 & lo1, g1, 0.0)
    ga1 = jnp.where(acc1 & ~lo1, g1, 0.0)
    gb0 = jnp.where(acc2 & lo2, g2, 0.0)
    gb1 = jnp.where(acc2 & ~lo2, g2, 0.0)
    dmask = 1.0 - jnp.maximum(acc1.astype(jnp.float32), acc2.astype(jnp.float32))

    meta_ref[...] = jnp.concatenate(
        [d1h0, d1h1, d2h0, d2h1, ga0, ga1, gb0, gb1, dmask,
         jnp.zeros((7, T), jnp.float32)], axis=0)

    imp = jnp.mean(probs, axis=1, keepdims=True)
    cnt = jnp.sum(gt, axis=1, keepdims=True)
    load = cnt / jnp.clip(jnp.sum(cnt), 1e-9, None)
    aux_ref[...] = jnp.reshape(float(E) * jnp.sum(imp * load), (1, 1))


def _router(x, wr):
    return pl.pallas_call(
        _router_body,
        out_shape=[
            jax.ShapeDtypeStruct((16, T), jnp.float32),
            jax.ShapeDtypeStruct((1, 1), jnp.float32),
        ],
    )(x, wr)


_NW = 32
_TPW = T // _NW
_CH = 64


def _dispatch_body(x_hbm, d0_hbm, d1_hbm, xe_hbm, xbuf, i0, i1, sem):
    wid = lax.axis_index("s") * 2 + lax.axis_index("c")
    for cch in range(_TPW // _CH):
        base = wid * _TPW + cch * _CH
        pltpu.sync_copy(x_hbm.at[pl.ds(base, _CH)], xbuf)
        pltpu.sync_copy(d0_hbm.at[pl.ds(base, _CH)], i0)
        pltpu.sync_copy(d1_hbm.at[pl.ds(base, _CH)], i1)
        pltpu.async_copy(xbuf, xe_hbm.at[i0], sem).wait()
        pltpu.async_copy(xbuf, xe_hbm.at[i1], sem).wait()


def _dispatch(x, d0, d1):
    mesh = plsc.VectorSubcoreMesh(core_axis_name="c", subcore_axis_name="s")
    return pl.kernel(
        _dispatch_body,
        mesh=mesh,
        out_type=jax.ShapeDtypeStruct((NRP, D), jnp.float32),
        scratch_types=[
            pltpu.VMEM((_CH, D), jnp.float32),
            pltpu.VMEM((_CH,), jnp.int32),
            pltpu.VMEM((_CH,), jnp.int32),
            pltpu.SemaphoreType.DMA,
        ],
    )(x, d0, d1)


def _mlp_body(xe_ref, w1_ref, b1_ref, w2_ref, b2_ref, ye_ref):
    ff = pl.program_id(1)

    @pl.when(ff == 0)
    def _init():
        ye_ref[...] = jnp.broadcast_to(b2_ref[0], (CAP, D))

    xv = xe_ref[...]
    w1 = w1_ref[0]
    h = lax.dot_general(xv, w1, (((1,), (1,)), ((), ())),
                        preferred_element_type=jnp.float32)
    h = h + b1_ref[0]
    h = 0.5 * h * (1.0 + lax.erf(h / _SQRT2))
    w2 = w2_ref[0]
    y = lax.dot_general(h, w2, (((1,), (1,)), ((), ())),
                        preferred_element_type=jnp.float32)
    ye_ref[...] += y


def _mlp(xe, w1, b1, w2, b2, eoff):
    return pl.pallas_call(
        _mlp_body,
        grid=(EH, NFT),
        in_specs=[
            pl.BlockSpec((CAP, D), lambda e, ff: (e, 0)),
            pl.BlockSpec((1, FT, D), lambda e, ff, o=eoff: (e + o, ff, 0)),
            pl.BlockSpec((1, 1, FT), lambda e, ff, o=eoff: (e + o, 0, ff)),
            pl.BlockSpec((1, D, FT), lambda e, ff, o=eoff: (e + o, 0, ff)),
            pl.BlockSpec((1, 1, D), lambda e, ff, o=eoff: (e + o, 0, 0)),
        ],
        out_specs=pl.BlockSpec((CAP, D), lambda e, ff: (e, 0)),
        out_shape=jax.ShapeDtypeStruct((NRP, D), jnp.float32),
    )(xe, w1, b1.reshape(E, 1, F), w2, b2.reshape(E, 1, D))


def _gather_body(ye_hbm, r0_hbm, r1_hbm, a_hbm, b_hbm, buf, idx, sem):
    wid = lax.axis_index("s") * 2 + lax.axis_index("c")
    for cch in range(_TPW // _CH):
        base = wid * _TPW + cch * _CH
        pltpu.sync_copy(r0_hbm.at[pl.ds(base, _CH)], idx)
        pltpu.async_copy(ye_hbm.at[idx], buf, sem).wait()
        pltpu.sync_copy(buf, a_hbm.at[pl.ds(base, _CH)])
        pltpu.sync_copy(r1_hbm.at[pl.ds(base, _CH)], idx)
        pltpu.async_copy(ye_hbm.at[idx], buf, sem).wait()
        pltpu.sync_copy(buf, b_hbm.at[pl.ds(base, _CH)])


def _gather2(ye, r0, r1):
    mesh = plsc.VectorSubcoreMesh(core_axis_name="c", subcore_axis_name="s")
    return pl.kernel(
        _gather_body,
        mesh=mesh,
        out_type=[
            jax.ShapeDtypeStruct((T, D), jnp.float32),
            jax.ShapeDtypeStruct((T, D), jnp.float32),
        ],
        scratch_types=[
            pltpu.VMEM((_CH, D), jnp.float32),
            pltpu.VMEM((_CH,), jnp.int32),
            pltpu.SemaphoreType.DMA,
        ],
    )(ye, r0, r1)


def _combine_body(x_ref, a0_ref, a1_ref, b0_ref, b1_ref,
                  g_refs, dm_ref, out_ref):
    za = jnp.zeros_like(x_ref[...])
    acc = dm_ref[...] * x_ref[...]
    for g_ref, src in zip(g_refs, (a0_ref, a1_ref, b0_ref, b1_ref)):
        g = g_ref[...]
        acc = acc + jnp.where(g > 0.0, g * src[...], za)
    out_ref[...] = acc


def _combine(x, a0, a1, b0, b1, gs, dm):
    nt = 32
    bt = T // nt
    row = pl.BlockSpec((bt, D), lambda i: (i, 0))
    col = pl.BlockSpec((bt, 1), lambda i: (i, 0))

    def body(x_ref, a0_ref, a1_ref, b0_ref, b1_ref,
             g0_ref, g1_ref, g2_ref, g3_ref, dm_ref, out_ref):
        _combine_body(x_ref, a0_ref, a1_ref, b0_ref, b1_ref,
                      (g0_ref, g1_ref, g2_ref, g3_ref), dm_ref, out_ref)

    return pl.pallas_call(
        body,
        grid=(nt,),
        in_specs=[row, row, row, row, row, col, col, col, col, col],
        out_specs=row,
        out_shape=jax.ShapeDtypeStruct((T, D), jnp.float32),
    )(x, a0, a1, b0, b1, *gs, dm)


def kernel(x, Wr, W1, b1, W2, b2):
    meta, aux = _router(x, Wr)
    d0h0 = meta[0].astype(jnp.int32)
    d0h1 = meta[1].astype(jnp.int32)
    d1h0 = meta[2].astype(jnp.int32)
    d1h1 = meta[3].astype(jnp.int32)
    xe0 = _dispatch(x, d0h0, d1h0)
    xe1 = _dispatch(x, d0h1, d1h1)
    ye0 = _mlp(xe0, W1, b1, W2, b2, 0)
    ye1 = _mlp(xe1, W1, b1, W2, b2, EH)
    a0, b0 = _gather2(ye0, d0h0, d1h0)
    a1, b1g = _gather2(ye1, d0h1, d1h1)
    gs = tuple(meta[4 + i].reshape(T, 1) for i in range(4))
    dm = meta[8].reshape(T, 1)
    out = _combine(x, a0, a1, b0, b1g, gs, dm)
    return out, aux[0, 0]


# speedup vs baseline: 3.1008x; 1.1116x over previous
import functools

import jax
import jax.numpy as jnp
from jax import lax
from jax.experimental import pallas as pl
from jax.experimental.pallas import tpu as pltpu
from jax.experimental.pallas import tpu_sc as plsc

T = 4096
D = 1024
F = 4096
E = 8
CAP = 1280
NR = E * CAP
PADR = CAP
NRP = NR + PADR
TRASH = NR
FT = 1024
NFT = F // FT

_SQRT2 = 1.4142135623730951


def _router_body(x_ref, wr_ref, meta_ref, didx_ref, aux_ref):
    xv = x_ref[...]
    wr = wr_ref[...]
    lg = lax.dot_general(wr, xv, (((1,), (1,)), ((), ())),
                         preferred_element_type=jnp.float32)
    m = jnp.max(lg, axis=0, keepdims=True)
    ex = jnp.exp(lg - m)
    probs = ex / jnp.sum(ex, axis=0, keepdims=True)

    iota = lax.broadcasted_iota(jnp.int32, (E, T), 0)
    m1 = jnp.max(probs, axis=0, keepdims=True)
    i1 = jnp.min(jnp.where(probs == m1, iota, E), axis=0, keepdims=True)
    pmask = iota == i1
    pm = jnp.where(pmask, -jnp.inf, probs)
    m2 = jnp.max(pm, axis=0, keepdims=True)
    i2 = jnp.min(jnp.where(pm == m2, iota, E), axis=0, keepdims=True)

    denom = jnp.clip(m1 + m2, 1e-9, None)
    g1 = m1 / denom
    g2 = m2 / denom

    oh1 = (iota == i1).astype(jnp.float32)
    oh2 = (iota == i2).astype(jnp.float32)
    gt = oh1 + oh2

    c = gt
    k = 1
    while k < T:
        shifted = jnp.concatenate(
            [jnp.zeros((E, k), jnp.float32), c[:, : T - k]], axis=1)
        c = c + shifted
        k *= 2
    excl = c - gt

    r1 = jnp.sum(oh1 * excl, axis=0, keepdims=True) + 1.0
    r2 = jnp.sum(oh2 * excl, axis=0, keepdims=True) + 1.0
    acc1 = r1 <= float(CAP)
    acc2 = r2 <= float(CAP)
    i1f = i1.astype(jnp.float32)
    i2f = i2.astype(jnp.float32)
    d1 = jnp.where(acc1, i1f * float(CAP) + r1 - 1.0, float(TRASH))
    d2 = jnp.where(acc2, i2f * float(CAP) + r2 - 1.0, float(TRASH))
    ga = jnp.where(acc1, g1, 0.0)
    gb = jnp.where(acc2, g2, 0.0)
    dmask = 1.0 - jnp.maximum(acc1.astype(jnp.float32), acc2.astype(jnp.float32))

    meta_ref[...] = jnp.concatenate(
        [ga, gb, dmask, jnp.zeros((5, T), jnp.float32)], axis=0)
    didx_ref[...] = jnp.concatenate([d1, d2], axis=0).astype(jnp.int32)

    imp = jnp.mean(probs, axis=1, keepdims=True)
    cnt = jnp.sum(gt, axis=1, keepdims=True)
    load = cnt / jnp.clip(jnp.sum(cnt), 1e-9, None)
    aux_ref[...] = jnp.reshape(float(E) * jnp.sum(imp * load), (1, 1))


def _router(x, wr):
    return pl.pallas_call(
        _router_body,
        out_shape=[
            jax.ShapeDtypeStruct((8, T), jnp.float32),
            jax.ShapeDtypeStruct((2, T), jnp.int32),
            jax.ShapeDtypeStruct((1, 1), jnp.float32),
        ],
    )(x, wr)


_NW = 32
_TPW = T // _NW
_CH = 64


def _dispatch_body(x_hbm, didx_hbm, xe_hbm, xbuf, i0, i1, sem):
    wid = lax.axis_index("s") * 2 + lax.axis_index("c")
    for cch in range(_TPW // _CH):
        base = wid * _TPW + cch * _CH
        pltpu.sync_copy(x_hbm.at[pl.ds(base, _CH)], xbuf)
        pltpu.sync_copy(didx_hbm.at[0, pl.ds(base, _CH)], i0)
        pltpu.sync_copy(didx_hbm.at[1, pl.ds(base, _CH)], i1)
        c0 = pltpu.async_copy(xbuf, xe_hbm.at[i0], sem)
        c1 = pltpu.async_copy(xbuf, xe_hbm.at[i1], sem)
        c0.wait()
        c1.wait()


def _dispatch(x, didx):
    mesh = plsc.VectorSubcoreMesh(core_axis_name="c", subcore_axis_name="s")
    return pl.kernel(
        _dispatch_body,
        mesh=mesh,
        out_type=jax.ShapeDtypeStruct((NRP, D), jnp.float32),
        scratch_types=[
            pltpu.VMEM((_CH, D), jnp.float32),
            pltpu.VMEM((_CH,), jnp.int32),
            pltpu.VMEM((_CH,), jnp.int32),
            pltpu.SemaphoreType.DMA,
        ],
    )(x, didx)


def _mlp_body(xe_ref, w1_ref, b1_ref, w2_ref, b2_ref, ye_ref):
    ff = pl.program_id(1)

    @pl.when(ff == 0)
    def _init():
        ye_ref[...] = jnp.broadcast_to(b2_ref[0], (CAP, D))

    xv = xe_ref[...]
    w1 = w1_ref[0]
    h = lax.dot_general(xv, w1, (((1,), (1,)), ((), ())),
                        preferred_element_type=jnp.float32)
    h = h + b1_ref[0]
    h = 0.5 * h * (1.0 + lax.erf(h / _SQRT2))
    w2 = w2_ref[0]
    y = lax.dot_general(h, w2, (((1,), (1,)), ((), ())),
                        preferred_element_type=jnp.float32)
    ye_ref[...] += y


def _mlp(xe, w1, b1, w2, b2):
    return pl.pallas_call(
        _mlp_body,
        grid=(E, NFT),
        in_specs=[
            pl.BlockSpec((CAP, D), lambda e, ff: (e, 0)),
            pl.BlockSpec((1, FT, D), lambda e, ff: (e, ff, 0)),
            pl.BlockSpec((1, 1, FT), lambda e, ff: (e, 0, ff)),
            pl.BlockSpec((1, D, FT), lambda e, ff: (e, 0, ff)),
            pl.BlockSpec((1, 1, D), lambda e, ff: (e, 0, 0)),
        ],
        out_specs=pl.BlockSpec((CAP, D), lambda e, ff: (e, 0)),
        out_shape=jax.ShapeDtypeStruct((NRP, D), jnp.float32),
    )(xe, w1, b1.reshape(E, 1, F), w2, b2.reshape(E, 1, D))


_CHG = 32


def _gather_body(ye_hbm, didx_hbm, a_hbm, b_hbm, bufa, bufb, i0, i1, sem):
    wid = lax.axis_index("s") * 2 + lax.axis_index("c")
    for cch in range(_TPW // _CHG):
        base = wid * _TPW + cch * _CHG
        pltpu.sync_copy(didx_hbm.at[0, pl.ds(base, _CHG)], i0)
        pltpu.sync_copy(didx_hbm.at[1, pl.ds(base, _CHG)], i1)
        c0 = pltpu.async_copy(ye_hbm.at[i0], bufa, sem)
        c1 = pltpu.async_copy(ye_hbm.at[i1], bufb, sem)
        c0.wait()
        c1.wait()
        pltpu.sync_copy(bufa, a_hbm.at[pl.ds(base, _CHG)])
        pltpu.sync_copy(bufb, b_hbm.at[pl.ds(base, _CHG)])


def _gather2(ye, didx):
    mesh = plsc.VectorSubcoreMesh(core_axis_name="c", subcore_axis_name="s")
    return pl.kernel(
        _gather_body,
        mesh=mesh,
        out_type=[
            jax.ShapeDtypeStruct((T, D), jnp.float32),
            jax.ShapeDtypeStruct((T, D), jnp.float32),
        ],
        scratch_types=[
            pltpu.VMEM((_CHG, D), jnp.float32),
            pltpu.VMEM((_CHG, D), jnp.float32),
            pltpu.VMEM((_CHG,), jnp.int32),
            pltpu.VMEM((_CHG,), jnp.int32),
            pltpu.SemaphoreType.DMA,
        ],
    )(ye, didx)


def _combine_body(x_ref, a_ref, b_ref, g0_ref, g1_ref, dm_ref, out_ref):
    g0 = g0_ref[...]
    g1 = g1_ref[...]
    za = jnp.zeros_like(a_ref[...])
    ca = jnp.where(g0 > 0.0, g0 * a_ref[...], za)
    cb = jnp.where(g1 > 0.0, g1 * b_ref[...], za)
    out_ref[...] = ca + cb + dm_ref[...] * x_ref[...]


def _combine(x, a, b, g0, g1, dm):
    nt = 32
    bt = T // nt
    return pl.pallas_call(
        _combine_body,
        grid=(nt,),
        in_specs=[
            pl.BlockSpec((bt, D), lambda i: (i, 0)),
            pl.BlockSpec((bt, D), lambda i: (i, 0)),
            pl.BlockSpec((bt, D), lambda i: (i, 0)),
            pl.BlockSpec((bt, 1), lambda i: (i, 0)),
            pl.BlockSpec((bt, 1), lambda i: (i, 0)),
            pl.BlockSpec((bt, 1), lambda i: (i, 0)),
        ],
        out_specs=pl.BlockSpec((bt, D), lambda i: (i, 0)),
        out_shape=jax.ShapeDtypeStruct((T, D), jnp.float32),
    )(x, a, b, g0, g1, dm)


def kernel(x, Wr, W1, b1, W2, b2):
    meta, didx, aux = _router(x, Wr)
    xe = _dispatch(x, didx)
    ye = _mlp(xe, W1, b1, W2, b2)
    a, b = _gather2(ye, didx)
    g0 = meta[0].reshape(T, 1)
    g1 = meta[1].reshape(T, 1)
    dm = meta[2].reshape(T, 1)
    out = _combine(x, a, b, g0, g1, dm)
    return out, aux[0, 0]
